# Initial kernel scaffold; baseline (speedup 1.0000x reference)
import jax
import jax.numpy as jnp
from jax.experimental import pallas as pl


def kernel(encode_hidden, target_batches, kb_memory, global_pointer, C_weight, W_ih, W_hh, b_ih, b_hh, W_proj, b_proj, W_mlp, b_mlp, max_target_length):
    raise NotImplementedError("write your pallas kernel here")


# speedup vs baseline: 3.2043x; 3.2043x over previous
import functools

import jax
import jax.numpy as jnp
from jax import lax
from jax.experimental import pallas as pl
from jax.experimental.pallas import tpu as pltpu
from jax.experimental.pallas import tpu_sc as plsc

_SOS_TOKEN = 2
_VOCAB_TILE = 2048


def _dot_t(a, b):
    return lax.dot_general(a, b, (((1,), (1,)), ((), ())),
                           preferred_element_type=jnp.float32)


def _sc_gather(table, idx):
    n = idx.shape[0]
    d = table.shape[1]
    info = plsc.get_sparse_core_info()
    num_workers = info.num_cores * info.num_subcores
    per_w = n // num_workers
    mesh = plsc.VectorSubcoreMesh(core_axis_name="c", subcore_axis_name="s")

    @functools.partial(
        pl.kernel, mesh=mesh,
        out_type=jax.ShapeDtypeStruct((n, d), jnp.float32),
        scratch_types=[
            pltpu.VMEM((per_w,), jnp.int32),
            pltpu.VMEM((per_w, d), jnp.float32),
            pltpu.SemaphoreType.DMA,
        ],
    )
    def gather_kernel(table_hbm, idx_hbm, out_hbm, idx_v, rows_v, sem):
        wid = lax.axis_index("s") * info.num_cores + lax.axis_index("c")
        base = wid * per_w
        pltpu.sync_copy(idx_hbm.at[pl.ds(base, per_w)], idx_v)
        pltpu.async_copy(table_hbm.at[idx_v], rows_v, sem).wait()
        pltpu.sync_copy(rows_v, out_hbm.at[pl.ds(base, per_w)])

    return gather_kernel(table, idx)


def _gru_chain_kernel(e_ref, eh_ref, kb_ref, gp_ref, wih_ref, whh_ref,
                      bih_ref, bhh_ref, wproj_ref, bproj_ref, wmlp_ref,
                      bmlp_ref, h_out_ref, ptr_ref, biased_ref, *, T, B, d):
    kb = kb_ref[...]
    gp = gp_ref[...]
    h = jnp.maximum(_dot_t(eh_ref[...], wproj_ref[...]) + bproj_ref[...], 0.0)
    for t in range(T):
        x = e_ref[pl.ds(t * B, B), :]
        gi = _dot_t(x, wih_ref[...]) + bih_ref[...]
        gh = _dot_t(h, whh_ref[...]) + bhh_ref[...]
        r = jax.nn.sigmoid(gi[:, 0:d] + gh[:, 0:d])
        z = jax.nn.sigmoid(gi[:, d:2 * d] + gh[:, d:2 * d])
        n = jnp.tanh(gi[:, 2 * d:3 * d] + r * gh[:, 2 * d:3 * d])
        h = (1.0 - z) * n + z * h
        h_out_ref[pl.ds(t * B, B), :] = h
        ptr_ref[t] = jnp.sum(h[:, None, :] * kb, axis=2) * gp
        mlp = _dot_t(h, wmlp_ref[...]) + bmlp_ref[...]
        biased_ref[t] = jnp.sum(mlp[:, None, :] * kb, axis=2) * gp


def _vocab_matmul_kernel(h_ref, c_ref, o_ref):
    o_ref[...] = _dot_t(h_ref[...], c_ref[...])


def kernel(encode_hidden, target_batches, kb_memory, global_pointer, C_weight,
           W_ih, W_hh, b_ih, b_hh, W_proj, b_proj, W_mlp, b_mlp,
           max_target_length):
    del max_target_length
    B, K, d = kb_memory.shape
    T = target_batches.shape[1]
    V = C_weight.shape[0]
    TB = T * B

    toks = jnp.concatenate(
        [jnp.full((1, B), _SOS_TOKEN, dtype=jnp.int32),
         target_batches[:, :T - 1].T.astype(jnp.int32)], axis=0
    ).reshape(TB)

    embeds = _sc_gather(C_weight, toks)

    gru = functools.partial(_gru_chain_kernel, T=T, B=B, d=d)
    h_all, ptr, biased = pl.pallas_call(
        gru,
        out_shape=(
            jax.ShapeDtypeStruct((TB, d), jnp.float32),
            jax.ShapeDtypeStruct((T, B, K), jnp.float32),
            jax.ShapeDtypeStruct((T, B, K), jnp.float32),
        ),
    )(embeds, encode_hidden, kb_memory, global_pointer, W_ih, W_hh,
      b_ih.reshape(1, -1), b_hh.reshape(1, -1), W_proj,
      b_proj.reshape(1, -1), W_mlp, b_mlp.reshape(1, -1))

    nv = pl.cdiv(V, _VOCAB_TILE)
    vocab = pl.pallas_call(
        _vocab_matmul_kernel,
        grid=(nv,),
        in_specs=[
            pl.BlockSpec((TB, d), lambda i: (0, 0)),
            pl.BlockSpec((_VOCAB_TILE, d), lambda i: (i, 0)),
        ],
        out_specs=pl.BlockSpec((TB, _VOCAB_TILE), lambda i: (0, i)),
        out_shape=jax.ShapeDtypeStruct((TB, V), jnp.float32),
    )(h_all, C_weight)

    return vocab.reshape(T, B, V), ptr, biased


# speedup vs baseline: 3.8521x; 1.2022x over previous
import functools

import jax
import jax.numpy as jnp
from jax import lax
from jax.experimental import pallas as pl
from jax.experimental.pallas import tpu as pltpu
from jax.experimental.pallas import tpu_sc as plsc

_SOS_TOKEN = 2
_VOCAB_TILE = 8192


def _dot_t(a, b):
    return lax.dot_general(a, b, (((1,), (1,)), ((), ())),
                           preferred_element_type=jnp.float32)


def _sc_gather(table, idx):
    n = idx.shape[0]
    d = table.shape[1]
    info = plsc.get_sparse_core_info()
    num_workers = info.num_cores * info.num_subcores
    per_w = n // num_workers
    mesh = plsc.VectorSubcoreMesh(core_axis_name="c", subcore_axis_name="s")

    @functools.partial(
        pl.kernel, mesh=mesh,
        out_type=jax.ShapeDtypeStruct((n, d), jnp.float32),
        scratch_types=[
            pltpu.VMEM((per_w,), jnp.int32),
            pltpu.VMEM((per_w, d), jnp.float32),
            pltpu.SemaphoreType.DMA,
        ],
    )
    def gather_kernel(table_hbm, idx_hbm, out_hbm, idx_v, rows_v, sem):
        wid = lax.axis_index("s") * info.num_cores + lax.axis_index("c")
        base = wid * per_w
        pltpu.sync_copy(idx_hbm.at[pl.ds(base, per_w)], idx_v)
        pltpu.async_copy(table_hbm.at[idx_v], rows_v, sem).wait()
        pltpu.sync_copy(rows_v, out_hbm.at[pl.ds(base, per_w)])

    return gather_kernel(table, idx)


def _gru_chain_kernel(e_ref, eh_ref, kb_ref, gp_ref, wih_ref, whh_ref,
                      bih_ref, bhh_ref, wproj_ref, bproj_ref, wmlp_ref,
                      bmlp_ref, h_out_ref, ptr_ref, biased_ref, *, T, B, d):
    kb = kb_ref[...]
    gp = gp_ref[...]
    h = jnp.maximum(_dot_t(eh_ref[...], wproj_ref[...]) + bproj_ref[...], 0.0)
    for t in range(T):
        x = e_ref[pl.ds(t * B, B), :]
        gi = _dot_t(x, wih_ref[...]) + bih_ref[...]
        gh = _dot_t(h, whh_ref[...]) + bhh_ref[...]
        r = jax.nn.sigmoid(gi[:, 0:d] + gh[:, 0:d])
        z = jax.nn.sigmoid(gi[:, d:2 * d] + gh[:, d:2 * d])
        n = jnp.tanh(gi[:, 2 * d:3 * d] + r * gh[:, 2 * d:3 * d])
        h = (1.0 - z) * n + z * h
        h_out_ref[pl.ds(t * B, B), :] = h
        ptr_ref[t] = jnp.sum(h[:, None, :] * kb, axis=2) * gp
        mlp = _dot_t(h, wmlp_ref[...]) + bmlp_ref[...]
        biased_ref[t] = jnp.sum(mlp[:, None, :] * kb, axis=2) * gp


def _vocab_matmul_kernel(h_ref, c_ref, o_ref):
    o_ref[...] = _dot_t(h_ref[...], c_ref[...])


def kernel(encode_hidden, target_batches, kb_memory, global_pointer, C_weight,
           W_ih, W_hh, b_ih, b_hh, W_proj, b_proj, W_mlp, b_mlp,
           max_target_length):
    del max_target_length
    B, K, d = kb_memory.shape
    T = target_batches.shape[1]
    V = C_weight.shape[0]
    TB = T * B

    toks = jnp.concatenate(
        [jnp.full((1, B), _SOS_TOKEN, dtype=jnp.int32),
         target_batches[:, :T - 1].T.astype(jnp.int32)], axis=0
    ).reshape(TB)

    embeds = _sc_gather(C_weight, toks)

    gru = functools.partial(_gru_chain_kernel, T=T, B=B, d=d)
    h_all, ptr, biased = pl.pallas_call(
        gru,
        out_shape=(
            jax.ShapeDtypeStruct((TB, d), jnp.float32),
            jax.ShapeDtypeStruct((T, B, K), jnp.float32),
            jax.ShapeDtypeStruct((T, B, K), jnp.float32),
        ),
    )(embeds, encode_hidden, kb_memory, global_pointer, W_ih, W_hh,
      b_ih.reshape(1, -1), b_hh.reshape(1, -1), W_proj,
      b_proj.reshape(1, -1), W_mlp, b_mlp.reshape(1, -1))

    nv = pl.cdiv(V, _VOCAB_TILE)
    vocab = pl.pallas_call(
        _vocab_matmul_kernel,
        grid=(nv,),
        in_specs=[
            pl.BlockSpec((TB, d), lambda i: (0, 0)),
            pl.BlockSpec((_VOCAB_TILE, d), lambda i: (i, 0)),
        ],
        out_specs=pl.BlockSpec((TB, _VOCAB_TILE), lambda i: (0, i)),
        out_shape=jax.ShapeDtypeStruct((TB, V), jnp.float32),
        compiler_params=pltpu.CompilerParams(
            dimension_semantics=("parallel",)),
    )(h_all, C_weight)

    return vocab.reshape(T, B, V), ptr, biased


# speedup vs baseline: 3.9293x; 1.0200x over previous
import functools

import jax
import jax.numpy as jnp
from jax import lax
from jax.experimental import pallas as pl
from jax.experimental.pallas import tpu as pltpu
from jax.experimental.pallas import tpu_sc as plsc

_SOS_TOKEN = 2
_VOCAB_TILE = 16384


def _dot_t(a, b):
    return lax.dot_general(a, b, (((1,), (1,)), ((), ())),
                           preferred_element_type=jnp.float32)


def _sc_gather(table, idx):
    n = idx.shape[0]
    d = table.shape[1]
    info = plsc.get_sparse_core_info()
    num_workers = info.num_cores * info.num_subcores
    per_w = n // num_workers
    mesh = plsc.VectorSubcoreMesh(core_axis_name="c", subcore_axis_name="s")

    @functools.partial(
        pl.kernel, mesh=mesh,
        out_type=jax.ShapeDtypeStruct((n, d), jnp.float32),
        scratch_types=[
            pltpu.VMEM((per_w,), jnp.int32),
            pltpu.VMEM((per_w, d), jnp.float32),
            pltpu.SemaphoreType.DMA,
        ],
    )
    def gather_kernel(table_hbm, idx_hbm, out_hbm, idx_v, rows_v, sem):
        wid = lax.axis_index("s") * info.num_cores + lax.axis_index("c")
        base = wid * per_w
        pltpu.sync_copy(idx_hbm.at[pl.ds(base, per_w)], idx_v)
        pltpu.async_copy(table_hbm.at[idx_v], rows_v, sem).wait()
        pltpu.sync_copy(rows_v, out_hbm.at[pl.ds(base, per_w)])

    return gather_kernel(table, idx)


def _gru_chain_kernel(e_ref, eh_ref, kb_ref, gp_ref, wih_ref, whh_ref,
                      bih_ref, bhh_ref, wproj_ref, bproj_ref, wmlp_ref,
                      bmlp_ref, h_out_ref, ptr_ref, biased_ref, *, T, B, d):
    kb = kb_ref[...]
    gp = gp_ref[...]
    h = jnp.maximum(_dot_t(eh_ref[...], wproj_ref[...]) + bproj_ref[...], 0.0)
    for t in range(T):
        x = e_ref[pl.ds(t * B, B), :]
        gi = _dot_t(x, wih_ref[...]) + bih_ref[...]
        gh = _dot_t(h, whh_ref[...]) + bhh_ref[...]
        r = jax.nn.sigmoid(gi[:, 0:d] + gh[:, 0:d])
        z = jax.nn.sigmoid(gi[:, d:2 * d] + gh[:, d:2 * d])
        n = jnp.tanh(gi[:, 2 * d:3 * d] + r * gh[:, 2 * d:3 * d])
        h = (1.0 - z) * n + z * h
        h_out_ref[pl.ds(t * B, B), :] = h
        ptr_ref[t] = jnp.sum(h[:, None, :] * kb, axis=2) * gp
        mlp = _dot_t(h, wmlp_ref[...]) + bmlp_ref[...]
        biased_ref[t] = jnp.sum(mlp[:, None, :] * kb, axis=2) * gp


def _vocab_matmul_kernel(h_ref, c_ref, o_ref):
    o_ref[...] = _dot_t(h_ref[...], c_ref[...])


def kernel(encode_hidden, target_batches, kb_memory, global_pointer, C_weight,
           W_ih, W_hh, b_ih, b_hh, W_proj, b_proj, W_mlp, b_mlp,
           max_target_length):
    del max_target_length
    B, K, d = kb_memory.shape
    T = target_batches.shape[1]
    V = C_weight.shape[0]
    TB = T * B

    toks = jnp.concatenate(
        [jnp.full((1, B), _SOS_TOKEN, dtype=jnp.int32),
         target_batches[:, :T - 1].T.astype(jnp.int32)], axis=0
    ).reshape(TB)

    embeds = _sc_gather(C_weight, toks)

    gru = functools.partial(_gru_chain_kernel, T=T, B=B, d=d)
    h_all, ptr, biased = pl.pallas_call(
        gru,
        out_shape=(
            jax.ShapeDtypeStruct((TB, d), jnp.float32),
            jax.ShapeDtypeStruct((T, B, K), jnp.float32),
            jax.ShapeDtypeStruct((T, B, K), jnp.float32),
        ),
    )(embeds, encode_hidden, kb_memory, global_pointer, W_ih, W_hh,
      b_ih.reshape(1, -1), b_hh.reshape(1, -1), W_proj,
      b_proj.reshape(1, -1), W_mlp, b_mlp.reshape(1, -1))

    nv = pl.cdiv(V, _VOCAB_TILE)
    vocab = pl.pallas_call(
        _vocab_matmul_kernel,
        grid=(nv,),
        in_specs=[
            pl.BlockSpec((TB, d), lambda i: (0, 0)),
            pl.BlockSpec((_VOCAB_TILE, d), lambda i: (i, 0)),
        ],
        out_specs=pl.BlockSpec((TB, _VOCAB_TILE), lambda i: (0, i)),
        out_shape=jax.ShapeDtypeStruct((TB, V), jnp.float32),
        compiler_params=pltpu.CompilerParams(
            dimension_semantics=("parallel",)),
    )(h_all, C_weight)

    return vocab.reshape(T, B, V), ptr, biased


# speedup vs baseline: 3.9337x; 1.0011x over previous
import functools

import jax
import jax.numpy as jnp
from jax import lax
from jax.experimental import pallas as pl
from jax.experimental.pallas import tpu as pltpu
from jax.experimental.pallas import tpu_sc as plsc

_SOS_TOKEN = 2
_VOCAB_TILE = 16384


def _dot_t(a, b):
    return lax.dot_general(a, b, (((1,), (1,)), ((), ())),
                           preferred_element_type=jnp.float32)


def _sc_gather(table, idx):
    n = idx.shape[0]
    d = table.shape[1]
    info = plsc.get_sparse_core_info()
    num_workers = info.num_cores * info.num_subcores
    per_w = n // num_workers
    mesh = plsc.VectorSubcoreMesh(core_axis_name="c", subcore_axis_name="s")

    @functools.partial(
        pl.kernel, mesh=mesh,
        out_type=jax.ShapeDtypeStruct((n, d), jnp.float32),
        scratch_types=[
            pltpu.VMEM((per_w,), jnp.int32),
            pltpu.VMEM((per_w, d), jnp.float32),
            pltpu.SemaphoreType.DMA,
        ],
    )
    def gather_kernel(table_hbm, idx_hbm, out_hbm, idx_v, rows_v, sem):
        wid = lax.axis_index("s") * info.num_cores + lax.axis_index("c")
        base = wid * per_w
        pltpu.sync_copy(idx_hbm.at[pl.ds(base, per_w)], idx_v)
        pltpu.async_copy(table_hbm.at[idx_v], rows_v, sem).wait()
        pltpu.sync_copy(rows_v, out_hbm.at[pl.ds(base, per_w)])

    return gather_kernel(table, idx)


def _gru_chain_kernel(e_ref, eh_ref, kb_ref, gp_ref, wih_ref, whh_ref,
                      bih_ref, bhh_ref, wproj_ref, bproj_ref, wmlp_ref,
                      bmlp_ref, h_out_ref, ptr_ref, biased_ref, *, T, B, d):
    kb = kb_ref[...]
    gp = gp_ref[...]
    h = jnp.maximum(_dot_t(eh_ref[...], wproj_ref[...]) + bproj_ref[...], 0.0)
    gi_all = _dot_t(e_ref[...], wih_ref[...]) + bih_ref[...]
    for t in range(T):
        gi = gi_all[t * B:(t + 1) * B]
        gh = _dot_t(h, whh_ref[...]) + bhh_ref[...]
        r = jax.nn.sigmoid(gi[:, 0:d] + gh[:, 0:d])
        z = jax.nn.sigmoid(gi[:, d:2 * d] + gh[:, d:2 * d])
        n = jnp.tanh(gi[:, 2 * d:3 * d] + r * gh[:, 2 * d:3 * d])
        h = (1.0 - z) * n + z * h
        h_out_ref[pl.ds(t * B, B), :] = h
        ptr_ref[t] = jnp.sum(h[:, None, :] * kb, axis=2) * gp
    mlp_all = _dot_t(h_out_ref[...], wmlp_ref[...]) + bmlp_ref[...]
    for t in range(T):
        biased_ref[t] = (
            jnp.sum(mlp_all[t * B:(t + 1) * B][:, None, :] * kb, axis=2) * gp)


def _vocab_matmul_kernel(h_ref, c_ref, o_ref):
    o_ref[...] = _dot_t(h_ref[...], c_ref[...])


def kernel(encode_hidden, target_batches, kb_memory, global_pointer, C_weight,
           W_ih, W_hh, b_ih, b_hh, W_proj, b_proj, W_mlp, b_mlp,
           max_target_length):
    del max_target_length
    B, K, d = kb_memory.shape
    T = target_batches.shape[1]
    V = C_weight.shape[0]
    TB = T * B

    toks = jnp.concatenate(
        [jnp.full((1, B), _SOS_TOKEN, dtype=jnp.int32),
         target_batches[:, :T - 1].T.astype(jnp.int32)], axis=0
    ).reshape(TB)

    embeds = _sc_gather(C_weight, toks)

    gru = functools.partial(_gru_chain_kernel, T=T, B=B, d=d)
    h_all, ptr, biased = pl.pallas_call(
        gru,
        out_shape=(
            jax.ShapeDtypeStruct((TB, d), jnp.float32),
            jax.ShapeDtypeStruct((T, B, K), jnp.float32),
            jax.ShapeDtypeStruct((T, B, K), jnp.float32),
        ),
    )(embeds, encode_hidden, kb_memory, global_pointer, W_ih, W_hh,
      b_ih.reshape(1, -1), b_hh.reshape(1, -1), W_proj,
      b_proj.reshape(1, -1), W_mlp, b_mlp.reshape(1, -1))

    nv = pl.cdiv(V, _VOCAB_TILE)
    vocab = pl.pallas_call(
        _vocab_matmul_kernel,
        grid=(nv,),
        in_specs=[
            pl.BlockSpec((TB, d), lambda i: (0, 0)),
            pl.BlockSpec((_VOCAB_TILE, d), lambda i: (i, 0)),
        ],
        out_specs=pl.BlockSpec((TB, _VOCAB_TILE), lambda i: (0, i)),
        out_shape=jax.ShapeDtypeStruct((TB, V), jnp.float32),
        compiler_params=pltpu.CompilerParams(
            dimension_semantics=("parallel",)),
    )(h_all, C_weight)

    return vocab.reshape(T, B, V), ptr, biased
